# baseline (speedup 1.0000x reference)
import jax
import jax.numpy as jnp
from jax.experimental import pallas as pl
from jax.experimental.pallas import tpu as pltpu


def _pick_bb(b):
    for bb in (4, 2, 1):
        if b % bb == 0:
            return bb
    return 1


def _chain_body(relus, modes, bb, chunk, tail_k, nsteps):
    n_layers = len(relus)

    def body(*refs):
        x_ref = refs[0]
        pos = 1
        layer_refs = []
        for mode in modes:
            if mode == "shared":
                layer_refs.append((refs[pos], None, refs[pos + 1]))
                pos += 2
            else:
                layer_refs.append((refs[pos], refs[pos + 1], refs[pos + 2]))
                pos += 3
        if tail_k:
            fc_refs = refs[pos:pos + 6]
            pos += 6
        o_ref = refs[pos]
        pool_ref = refs[pos + 1] if tail_k else None
        g = pl.program_id(0)

        for i in range(bb):
            h = x_ref[i]
            for li in range(n_layers):
                w_ref, aux_ref, s_ref = layer_refs[li]
                if modes[li] == "shared":
                    w = w_ref[...]
                elif modes[li] == "fold1":
                    w = jnp.dot(w_ref[...], aux_ref[i],
                                preferred_element_type=jnp.float32
                                ).astype(jnp.bfloat16)
                else:
                    w = jax.lax.dot_general(
                        w_ref[...], aux_ref[i].astype(jnp.bfloat16),
                        (((1,), (1,)), ((), ())),
                        preferred_element_type=jnp.float32).astype(jnp.bfloat16)
                if li < n_layers - 1:
                    y = jnp.dot(w, h, preferred_element_type=jnp.float32) + s_ref[...]
                    if relus[li]:
                        y = jnp.maximum(y, 0.0)
                    h = y.astype(jnp.bfloat16)
                else:
                    n = h.shape[1]
                    m = None
                    for c0 in range(0, n, chunk):
                        yc = jnp.dot(w, h[:, c0:c0 + chunk],
                                     preferred_element_type=jnp.float32)
                        for l0 in range(0, chunk, 128):
                            blk = yc[:, l0:l0 + 128]
                            m = blk if m is None else jnp.maximum(m, blk)
                    row = jnp.max(jnp.transpose(m), axis=0, keepdims=True) + s_ref[...]
                    if relus[li]:
                        row = jnp.maximum(row, 0.0)
                    if tail_k:
                        pool_ref[pl.ds(g * bb + i, 1), :] = row
                    else:
                        o_ref[0, i] = row[0]

        if tail_k:
            @pl.when(g == nsteps - 1)
            def _tail():
                w1, s1, w2, s2, w3, s3 = fc_refs
                hh = pool_ref[...].astype(jnp.bfloat16)
                hh = jnp.maximum(
                    jnp.dot(hh, w1[...], preferred_element_type=jnp.float32)
                    + s1[...], 0.0).astype(jnp.bfloat16)
                hh = jnp.maximum(
                    jnp.dot(hh, w2[...], preferred_element_type=jnp.float32)
                    + s2[...], 0.0).astype(jnp.bfloat16)
                yy = jnp.dot(hh, w3[...], preferred_element_type=jnp.float32) + s3[...]
                j = jax.lax.broadcasted_iota(jnp.int32, yy.shape, 1)
                o_ref[...] = yy + jnp.where(j % (tail_k + 1) == 0, 1.0, 0.0)

    return body


def _chain(x_cf, layers, relus, modes, tail=None, tail_k=0):
    b, cin0, n = x_cf.shape
    if n % 128:
        x_cf = jnp.pad(x_cf, ((0, 0), (0, 0), (0, 128 - n % 128)), mode="edge")
        n = x_cf.shape[2]
    bb = _pick_bb(b)
    nsteps = b // bb
    chunk = 512
    while n % chunk:
        chunk //= 2

    in_specs = [pl.BlockSpec((bb, cin0, n), lambda g: (g, 0, 0))]
    args = [x_cf]
    nl = len(layers)
    for li, (w, aux, sh) in enumerate(layers):
        co = w.shape[0]
        in_specs.append(pl.BlockSpec(w.shape, lambda g: (0, 0)))
        args.append(w)
        if aux is not None:
            in_specs.append(pl.BlockSpec((bb,) + aux.shape[1:], lambda g: (g, 0, 0)))
            args.append(aux)
        if li == nl - 1:
            in_specs.append(pl.BlockSpec((1, co), lambda g: (0, 0)))
            args.append(sh.reshape(1, co))
        else:
            in_specs.append(pl.BlockSpec((co, 1), lambda g: (0, 0)))
            args.append(sh)

    scratch = []
    if tail is not None:
        for w, sh in tail:
            in_specs.append(pl.BlockSpec(w.shape, lambda g: (0, 0)))
            args.append(w)
            in_specs.append(pl.BlockSpec(sh.shape, lambda g: (0, 0)))
            args.append(sh)
        c_last = tail_k * tail_k
        out_specs = pl.BlockSpec((b, c_last), lambda g: (0, 0))
        out_shape = jax.ShapeDtypeStruct((b, c_last), jnp.float32)
        scratch = [pltpu.VMEM((b, layers[-1][0].shape[0]), jnp.float32)]
    else:
        c_last = layers[-1][0].shape[0]
        out_specs = pl.BlockSpec((1, bb, c_last), lambda g: (g, 0, 0))
        out_shape = jax.ShapeDtypeStruct((nsteps, bb, c_last), jnp.float32)

    out = pl.pallas_call(
        _chain_body(tuple(relus), tuple(modes), bb, chunk, tail_k, nsteps),
        grid=(nsteps,),
        in_specs=in_specs,
        out_specs=out_specs,
        out_shape=out_shape,
        scratch_shapes=scratch,
        compiler_params=pltpu.CompilerParams(
            dimension_semantics=("arbitrary",),
            vmem_limit_bytes=64 * 1024 * 1024),
    )(*args)
    if tail is None:
        out = out.reshape(b, c_last)
    return out


def kernel(x,
           stn3d_conv1_k0, stn3d_conv1_k1, stn3d_conv2_k0, stn3d_conv2_k1,
           stn3d_conv3_k0, stn3d_conv3_k1, stn3d_fc1_k0, stn3d_fc1_k1,
           stn3d_fc2_k0, stn3d_fc2_k1, stn3d_fc3_k0, stn3d_fc3_k1,
           fstn_conv1_k0, fstn_conv1_k1, fstn_conv2_k0, fstn_conv2_k1,
           fstn_conv3_k0, fstn_conv3_k1, fstn_fc1_k0, fstn_fc1_k1,
           fstn_fc2_k0, fstn_fc2_k1, fstn_fc3_k0, fstn_fc3_k1,
           conv1_w, conv1_sh, conv2_w, conv2_sh, conv3_k0, conv3_sh):
    b, c, _ = x.shape
    x_bf = x.astype(jnp.bfloat16)
    w1cf = jnp.transpose(conv1_w).astype(jnp.bfloat16)
    w2cf = jnp.transpose(conv2_w).astype(jnp.bfloat16)

    trans = _chain(
        x_bf,
        [(stn3d_conv1_k0, None, stn3d_conv1_k1),
         (stn3d_conv2_k0, None, stn3d_conv2_k1),
         (stn3d_conv3_k0, None, stn3d_conv3_k1)],
        relus=(True, True, True), modes=("shared",) * 3,
        tail=((stn3d_fc1_k0, stn3d_fc1_k1), (stn3d_fc2_k0, stn3d_fc2_k1),
              (stn3d_fc3_k0, stn3d_fc3_k1)), tail_k=3)

    t33t = jnp.transpose(trans.reshape(b, 3, 3), (0, 2, 1))
    z3 = jnp.zeros((b, 3, c - 3), jnp.float32)
    bd = jnp.concatenate([
        jnp.concatenate([t33t, z3], axis=2),
        jnp.concatenate([jnp.zeros((b, c - 3, 3), jnp.float32),
                         jnp.broadcast_to(jnp.eye(c - 3, dtype=jnp.float32)[None],
                                          (b, c - 3, c - 3))], axis=2),
    ], axis=1).astype(jnp.bfloat16)

    tf_flat = _chain(
        x_bf,
        [(w1cf, bd, conv1_sh),
         (fstn_conv1_k0, None, fstn_conv1_k1),
         (fstn_conv2_k0, None, fstn_conv2_k1),
         (fstn_conv3_k0, None, fstn_conv3_k1)],
        relus=(True, True, True, True), modes=("fold1", "shared", "shared", "shared"),
        tail=((fstn_fc1_k0, fstn_fc1_k1), (fstn_fc2_k0, fstn_fc2_k1),
              (fstn_fc3_k0, fstn_fc3_k1)), tail_k=64)
    tf = tf_flat.reshape(b, 64, 64)

    return _chain(
        x_bf,
        [(w1cf, bd, conv1_sh),
         (w2cf, tf, conv2_sh),
         (conv3_k0, None, conv3_sh)],
        relus=(True, True, False), modes=("fold1", "fold2", "shared"))


# speedup vs baseline: 1.0399x; 1.0399x over previous
import functools

import jax
import jax.numpy as jnp
from jax.experimental import pallas as pl
from jax.experimental.pallas import tpu as pltpu


def _pick_bb(b):
    for bb in (4, 2, 1):
        if b % bb == 0:
            return bb
    return 1


def _chain_body(relus, per_batch, bb, chunk):
    n_layers = len(relus)

    def body(*refs):
        x_ref = refs[0]
        o_ref = refs[-1]
        for i in range(bb):
            h = x_ref[i]
            for li in range(n_layers - 1):
                w_ref = refs[1 + 2 * li]
                s_ref = refs[2 + 2 * li]
                w = w_ref[i] if per_batch[li] else w_ref[...]
                y = jnp.dot(w, h, preferred_element_type=jnp.float32) + s_ref[...]
                if relus[li]:
                    y = jnp.maximum(y, 0.0)
                h = y.astype(jnp.bfloat16)
            wl_ref = refs[2 * n_layers - 1]
            sl_ref = refs[2 * n_layers]
            wl = wl_ref[i] if per_batch[-1] else wl_ref[...]
            n = h.shape[1]
            m = None
            for c0 in range(0, n, chunk):
                yc = jnp.dot(wl, h[:, c0:c0 + chunk],
                             preferred_element_type=jnp.float32)
                for l0 in range(0, chunk, 128):
                    blk = yc[:, l0:l0 + 128]
                    m = blk if m is None else jnp.maximum(m, blk)
            row = jnp.max(jnp.transpose(m), axis=0, keepdims=True) + sl_ref[...]
            if relus[-1]:
                row = jnp.maximum(row, 0.0)
            o_ref[0, i] = row[0]

    return body


def _chain_maxpool(x_cf, layers, relus, per_batch):
    b, cin0, n = x_cf.shape
    if n % 128:
        x_cf = jnp.pad(x_cf, ((0, 0), (0, 0), (0, 128 - n % 128)), mode="edge")
        n = x_cf.shape[2]
    bb = _pick_bb(b)
    chunk = 512
    while n % chunk:
        chunk //= 2

    in_specs = [pl.BlockSpec((bb, cin0, n), lambda g: (g, 0, 0))]
    args = [x_cf]
    nl = len(layers)
    for li, (w, sh) in enumerate(layers):
        last = li == nl - 1
        if per_batch[li]:
            _, co, ci = w.shape
            in_specs.append(pl.BlockSpec((bb, co, ci), lambda g: (g, 0, 0)))
        else:
            co, ci = w.shape
            in_specs.append(pl.BlockSpec((co, ci), lambda g: (0, 0)))
        args.append(w)
        if last:
            in_specs.append(pl.BlockSpec((1, co), lambda g: (0, 0)))
            args.append(sh.reshape(1, co))
        else:
            in_specs.append(pl.BlockSpec((co, 1), lambda g: (0, 0)))
            args.append(sh)

    c_last = layers[-1][0].shape[-2]
    out = pl.pallas_call(
        _chain_body(tuple(relus), tuple(per_batch), bb, chunk),
        grid=(b // bb,),
        in_specs=in_specs,
        out_specs=pl.BlockSpec((1, bb, c_last), lambda g: (g, 0, 0)),
        out_shape=jax.ShapeDtypeStruct((b // bb, bb, c_last), jnp.float32),
        compiler_params=pltpu.CompilerParams(
            dimension_semantics=("parallel",),
            vmem_limit_bytes=64 * 1024 * 1024),
    )(*args)
    return out.reshape(b, c_last)


def _fc_body(p_ref, w1_ref, s1_ref, w2_ref, s2_ref, w3_ref, s3_ref, o_ref, *, k):
    h = p_ref[...].astype(jnp.bfloat16)
    h = jnp.maximum(
        jnp.dot(h, w1_ref[...], preferred_element_type=jnp.float32) + s1_ref[...],
        0.0).astype(jnp.bfloat16)
    h = jnp.maximum(
        jnp.dot(h, w2_ref[...], preferred_element_type=jnp.float32) + s2_ref[...],
        0.0).astype(jnp.bfloat16)
    y = jnp.dot(h, w3_ref[...], preferred_element_type=jnp.float32) + s3_ref[...]
    j = jax.lax.broadcasted_iota(jnp.int32, y.shape, 1)
    o_ref[...] = y + jnp.where(j % (k + 1) == 0, 1.0, 0.0).astype(jnp.float32)


def _fc_tail(pooled, fc1, fc2, fc3, k):
    b, d = pooled.shape
    ws = [fc1[0], fc1[1], fc2[0], fc2[1], fc3[0], fc3[1]]
    return pl.pallas_call(
        functools.partial(_fc_body, k=k),
        out_shape=jax.ShapeDtypeStruct((b, k * k), jnp.float32),
    )(pooled, *ws)


def kernel(x,
           stn3d_conv1_k0, stn3d_conv1_k1, stn3d_conv2_k0, stn3d_conv2_k1,
           stn3d_conv3_k0, stn3d_conv3_k1, stn3d_fc1_k0, stn3d_fc1_k1,
           stn3d_fc2_k0, stn3d_fc2_k1, stn3d_fc3_k0, stn3d_fc3_k1,
           fstn_conv1_k0, fstn_conv1_k1, fstn_conv2_k0, fstn_conv2_k1,
           fstn_conv3_k0, fstn_conv3_k1, fstn_fc1_k0, fstn_fc1_k1,
           fstn_fc2_k0, fstn_fc2_k1, fstn_fc3_k0, fstn_fc3_k1,
           conv1_w, conv1_sh, conv2_w, conv2_sh, conv3_k0, conv3_sh):
    b, c, _ = x.shape
    x_bf = x.astype(jnp.bfloat16)

    pooled = _chain_maxpool(
        x_bf,
        [(stn3d_conv1_k0, stn3d_conv1_k1), (stn3d_conv2_k0, stn3d_conv2_k1),
         (stn3d_conv3_k0, stn3d_conv3_k1)],
        relus=(True, True, True), per_batch=(False, False, False))
    trans = _fc_tail(pooled, (stn3d_fc1_k0, stn3d_fc1_k1),
                     (stn3d_fc2_k0, stn3d_fc2_k1),
                     (stn3d_fc3_k0, stn3d_fc3_k1), k=3).reshape(b, 3, 3)

    co1 = conv1_w.shape[1]
    w1p_t = jnp.concatenate([
        jnp.einsum("jo,bcj->boc", conv1_w[:3], trans),
        jnp.broadcast_to(jnp.transpose(conv1_w[3:])[None], (b, co1, c - 3)),
    ], axis=2).astype(jnp.bfloat16)

    pooled_f = _chain_maxpool(
        x_bf,
        [(w1p_t, conv1_sh), (fstn_conv1_k0, fstn_conv1_k1),
         (fstn_conv2_k0, fstn_conv2_k1), (fstn_conv3_k0, fstn_conv3_k1)],
        relus=(True, True, True, True), per_batch=(True, False, False, False))
    trans_feat = _fc_tail(pooled_f, (fstn_fc1_k0, fstn_fc1_k1),
                          (fstn_fc2_k0, fstn_fc2_k1),
                          (fstn_fc3_k0, fstn_fc3_k1), k=64).reshape(b, 64, 64)

    w2p_t = jnp.einsum("jo,bij->boi", conv2_w,
                       trans_feat).astype(jnp.bfloat16)

    return _chain_maxpool(
        x_bf,
        [(w1p_t, conv1_sh), (w2p_t, conv2_sh), (conv3_k0, conv3_sh)],
        relus=(True, True, False), per_batch=(True, True, False))


# speedup vs baseline: 1.0547x; 1.0143x over previous
import functools

import jax
import jax.numpy as jnp
from jax.experimental import pallas as pl
from jax.experimental.pallas import tpu as pltpu


def _pick_bb(b):
    for bb in (4, 2, 1):
        if b % bb == 0:
            return bb
    return 1


def _chain_body(relus, per_batch, bb, chunk, export_l0):
    n_layers = len(relus)

    def body(*refs):
        x_ref = refs[0]
        if export_l0:
            o_ref, h1_ref = refs[-2], refs[-1]
        else:
            o_ref = refs[-1]
        for i in range(bb):
            h = x_ref[i]
            for li in range(n_layers - 1):
                w_ref = refs[1 + 2 * li]
                s_ref = refs[2 + 2 * li]
                w = w_ref[i] if per_batch[li] else w_ref[...]
                y = jnp.dot(w, h, preferred_element_type=jnp.float32) + s_ref[...]
                if relus[li]:
                    y = jnp.maximum(y, 0.0)
                h = y.astype(jnp.bfloat16)
                if li == 0 and export_l0:
                    h1_ref[i] = h
            wl_ref = refs[2 * n_layers - 1]
            sl_ref = refs[2 * n_layers]
            wl = wl_ref[i] if per_batch[-1] else wl_ref[...]
            n = h.shape[1]
            m = None
            for c0 in range(0, n, chunk):
                yc = jnp.dot(wl, h[:, c0:c0 + chunk],
                             preferred_element_type=jnp.float32)
                for l0 in range(0, chunk, 128):
                    blk = yc[:, l0:l0 + 128]
                    m = blk if m is None else jnp.maximum(m, blk)
            row = jnp.max(jnp.transpose(m), axis=0, keepdims=True) + sl_ref[...]
            if relus[-1]:
                row = jnp.maximum(row, 0.0)
            o_ref[0, i] = row[0]

    return body


def _chain_maxpool(x_cf, layers, relus, per_batch, export_l0=False):
    b, cin0, n = x_cf.shape
    if n % 128:
        x_cf = jnp.pad(x_cf, ((0, 0), (0, 0), (0, 128 - n % 128)), mode="edge")
        n = x_cf.shape[2]
    bb = _pick_bb(b)
    chunk = 512
    while n % chunk:
        chunk //= 2

    in_specs = [pl.BlockSpec((bb, cin0, n), lambda g: (g, 0, 0))]
    args = [x_cf]
    nl = len(layers)
    for li, (w, sh) in enumerate(layers):
        last = li == nl - 1
        if per_batch[li]:
            _, co, ci = w.shape
            in_specs.append(pl.BlockSpec((bb, co, ci), lambda g: (g, 0, 0)))
        else:
            co, ci = w.shape
            in_specs.append(pl.BlockSpec((co, ci), lambda g: (0, 0)))
        args.append(w)
        if last:
            in_specs.append(pl.BlockSpec((1, co), lambda g: (0, 0)))
            args.append(sh.reshape(1, co))
        else:
            in_specs.append(pl.BlockSpec((co, 1), lambda g: (0, 0)))
            args.append(sh)

    c_last = layers[-1][0].shape[-2]
    out_specs = [pl.BlockSpec((1, bb, c_last), lambda g: (g, 0, 0))]
    out_shape = [jax.ShapeDtypeStruct((b // bb, bb, c_last), jnp.float32)]
    if export_l0:
        co0 = layers[0][0].shape[-2]
        out_specs.append(pl.BlockSpec((bb, co0, n), lambda g: (g, 0, 0)))
        out_shape.append(jax.ShapeDtypeStruct((b, co0, n), jnp.bfloat16))
    out = pl.pallas_call(
        _chain_body(tuple(relus), tuple(per_batch), bb, chunk, export_l0),
        grid=(b // bb,),
        in_specs=in_specs,
        out_specs=out_specs,
        out_shape=out_shape,
        compiler_params=pltpu.CompilerParams(
            dimension_semantics=("parallel",),
            vmem_limit_bytes=64 * 1024 * 1024),
    )(*args)
    pooled = out[0].reshape(b, c_last)
    return (pooled, out[1]) if export_l0 else pooled


def _fc_body(p_ref, w1_ref, s1_ref, w2_ref, s2_ref, w3_ref, s3_ref, o_ref, *, k):
    h = p_ref[...].astype(jnp.bfloat16)
    h = jnp.maximum(
        jnp.dot(h, w1_ref[...], preferred_element_type=jnp.float32) + s1_ref[...],
        0.0).astype(jnp.bfloat16)
    h = jnp.maximum(
        jnp.dot(h, w2_ref[...], preferred_element_type=jnp.float32) + s2_ref[...],
        0.0).astype(jnp.bfloat16)
    y = jnp.dot(h, w3_ref[...], preferred_element_type=jnp.float32) + s3_ref[...]
    j = jax.lax.broadcasted_iota(jnp.int32, y.shape, 1)
    o_ref[...] = y + jnp.where(j % (k + 1) == 0, 1.0, 0.0).astype(jnp.float32)


def _fc_tail(pooled, fc1, fc2, fc3, k):
    b, d = pooled.shape
    ws = [fc1[0], fc1[1], fc2[0], fc2[1], fc3[0], fc3[1]]
    return pl.pallas_call(
        functools.partial(_fc_body, k=k),
        out_shape=jax.ShapeDtypeStruct((b, k * k), jnp.float32),
    )(pooled, *ws)


def kernel(x,
           stn3d_conv1_k0, stn3d_conv1_k1, stn3d_conv2_k0, stn3d_conv2_k1,
           stn3d_conv3_k0, stn3d_conv3_k1, stn3d_fc1_k0, stn3d_fc1_k1,
           stn3d_fc2_k0, stn3d_fc2_k1, stn3d_fc3_k0, stn3d_fc3_k1,
           fstn_conv1_k0, fstn_conv1_k1, fstn_conv2_k0, fstn_conv2_k1,
           fstn_conv3_k0, fstn_conv3_k1, fstn_fc1_k0, fstn_fc1_k1,
           fstn_fc2_k0, fstn_fc2_k1, fstn_fc3_k0, fstn_fc3_k1,
           conv1_w, conv1_sh, conv2_w, conv2_sh, conv3_k0, conv3_sh):
    b, c, _ = x.shape
    x_bf = x.astype(jnp.bfloat16)

    pooled = _chain_maxpool(
        x_bf,
        [(stn3d_conv1_k0, stn3d_conv1_k1), (stn3d_conv2_k0, stn3d_conv2_k1),
         (stn3d_conv3_k0, stn3d_conv3_k1)],
        relus=(True, True, True), per_batch=(False, False, False))
    trans = _fc_tail(pooled, (stn3d_fc1_k0, stn3d_fc1_k1),
                     (stn3d_fc2_k0, stn3d_fc2_k1),
                     (stn3d_fc3_k0, stn3d_fc3_k1), k=3).reshape(b, 3, 3)

    co1 = conv1_w.shape[1]
    w1p_t = jnp.concatenate([
        jnp.einsum("jo,bcj->boc", conv1_w[:3], trans),
        jnp.broadcast_to(jnp.transpose(conv1_w[3:])[None], (b, co1, c - 3)),
    ], axis=2).astype(jnp.bfloat16)

    pooled_f, h1p = _chain_maxpool(
        x_bf,
        [(w1p_t, conv1_sh), (fstn_conv1_k0, fstn_conv1_k1),
         (fstn_conv2_k0, fstn_conv2_k1), (fstn_conv3_k0, fstn_conv3_k1)],
        relus=(True, True, True, True), per_batch=(True, False, False, False),
        export_l0=True)
    trans_feat = _fc_tail(pooled_f, (fstn_fc1_k0, fstn_fc1_k1),
                          (fstn_fc2_k0, fstn_fc2_k1),
                          (fstn_fc3_k0, fstn_fc3_k1), k=64).reshape(b, 64, 64)

    w2p_t = jnp.einsum("jo,bij->boi", conv2_w,
                       trans_feat).astype(jnp.bfloat16)

    return _chain_maxpool(
        h1p,
        [(w2p_t, conv2_sh), (conv3_k0, conv3_sh)],
        relus=(True, False), per_batch=(True, False))
